# baseline (speedup 1.0000x reference)
import functools
import math

import jax
import jax.numpy as jnp
from jax import lax
from jax.experimental import pallas as pl
from jax.experimental.pallas import tpu as pltpu
from jax.experimental.pallas import tpu_sc as plsc

D_MODEL = 64
SCALE = math.sqrt(D_MODEL)
NUM_CORES = 2
NUM_SUBCORES = 16
NUM_WORKERS = NUM_CORES * NUM_SUBCORES
CHUNK = 800
LANES = 16


def _make_kernel(B):
    assert B % (8 * NUM_WORKERS) == 0
    b_per_w = B // NUM_WORKERS
    assert b_per_w % CHUNK == 0
    n_chunks = b_per_w // CHUNK

    mesh = plsc.VectorSubcoreMesh(
        core_axis_name="c",
        subcore_axis_name="s",
        num_cores=NUM_CORES,
        num_subcores=NUM_SUBCORES,
    )

    @functools.partial(
        pl.kernel,
        mesh=mesh,
        compiler_params=pltpu.CompilerParams(use_tc_tiling_on_sc=False),
        out_type=jax.ShapeDtypeStruct((B, D_MODEL), jnp.float32),
        scratch_types=[
            pltpu.VMEM((CHUNK,), jnp.int32),
            pltpu.VMEM((CHUNK, D_MODEL), jnp.float32),
            pltpu.SemaphoreType.DMA,
        ],
    )
    def emb(idx_hbm, table_hbm, out_hbm, idx_v, rows_v, sem):
        wid = lax.axis_index("s") * NUM_CORES + lax.axis_index("c")
        base = wid * b_per_w

        def chunk_body(g, carry):
            off = base + g * CHUNK
            pltpu.sync_copy(idx_hbm.at[pl.ds(off, CHUNK)], idx_v)
            pltpu.async_copy(table_hbm.at[idx_v], rows_v, sem).wait()

            def row_body(i, c):
                for j in range(D_MODEL // LANES):
                    sl = pl.ds(j * LANES, LANES)
                    rows_v[i, sl] = rows_v[i, sl] * SCALE
                return c

            lax.fori_loop(0, CHUNK, row_body, 0, unroll=4)
            pltpu.sync_copy(rows_v, out_hbm.at[pl.ds(off, CHUNK)])
            return carry

        lax.fori_loop(0, n_chunks, chunk_body, 0)

    return emb


def kernel(input, table):
    B0, B1 = input.shape
    B = B0 * B1
    idx = input.reshape(B).astype(jnp.int32)
    out = _make_kernel(B)(idx, table)
    return out.reshape(B0, B1, D_MODEL)


# speedup vs baseline: 1.0682x; 1.0682x over previous
import functools
import math

import jax
import jax.numpy as jnp
from jax import lax
from jax.experimental import pallas as pl
from jax.experimental.pallas import tpu as pltpu
from jax.experimental.pallas import tpu_sc as plsc

D_MODEL = 64
SCALE = math.sqrt(D_MODEL)
NUM_CORES = 2
NUM_SUBCORES = 16
NUM_WORKERS = NUM_CORES * NUM_SUBCORES
CHUNK = 400
NBUF = 4
LANES = 16


def _make_kernel(B):
    assert B % (8 * NUM_WORKERS) == 0
    b_per_w = B // NUM_WORKERS
    assert b_per_w % (CHUNK * NBUF) == 0
    n_chunks = b_per_w // CHUNK

    mesh = plsc.VectorSubcoreMesh(
        core_axis_name="c",
        subcore_axis_name="s",
        num_cores=NUM_CORES,
        num_subcores=NUM_SUBCORES,
    )

    @functools.partial(
        pl.kernel,
        mesh=mesh,
        compiler_params=pltpu.CompilerParams(use_tc_tiling_on_sc=False),
        out_type=jax.ShapeDtypeStruct((B, D_MODEL), jnp.float32),
        scratch_types=[
            pltpu.VMEM((b_per_w,), jnp.int32),
        ]
        + [pltpu.VMEM((CHUNK, D_MODEL), jnp.float32)] * NBUF
        + [pltpu.SemaphoreType.DMA] * (2 * NBUF),
    )
    def emb(idx_hbm, table_hbm, out_hbm, idx_all,
            r0, r1, r2, r3, g0, g1, g2, g3, s0, s1, s2, s3):
        rows = (r0, r1, r2, r3)
        gsem = (g0, g1, g2, g3)
        ssem = (s0, s1, s2, s3)
        wid = lax.axis_index("s") * NUM_CORES + lax.axis_index("c")
        base = wid * b_per_w

        pltpu.sync_copy(idx_hbm.at[pl.ds(base, b_per_w)], idx_all)

        def g_copy(g, b):
            return pltpu.make_async_copy(
                table_hbm.at[idx_all.at[pl.ds(g * CHUNK, CHUNK)]],
                rows[b], gsem[b])

        def s_copy(g, b):
            return pltpu.make_async_copy(
                rows[b], out_hbm.at[pl.ds(base + g * CHUNK, CHUNK)], ssem[b])

        def scale(b):
            r = rows[b]

            @plsc.parallel_loop(0, CHUNK, step=1, unroll=8)
            def _(i):
                for j in range(D_MODEL // LANES):
                    sl = pl.ds(j * LANES, LANES)
                    r[i, sl] = r[i, sl] * SCALE

        def process(g, b, prefetch_g=None, prefetch_b=None, sswait_g=None):
            if sswait_g is not None:
                s_copy(sswait_g, prefetch_b).wait()
            if prefetch_g is not None:
                g_copy(prefetch_g, prefetch_b).start()
            g_copy(g, b).wait()
            scale(b)
            s_copy(g, b).start()

        g_copy(0, 0).start()
        g_copy(1, 1).start()
        process(0, 0, prefetch_g=2, prefetch_b=2)
        process(1, 1, prefetch_g=3, prefetch_b=3)

        def group(i, carry):
            gbase = 2 + i * NBUF
            for j in range(NBUF):
                g = gbase + j
                b = (2 + j) % NBUF
                process(g, b, prefetch_g=g + 2, prefetch_b=(b + 2) % NBUF,
                        sswait_g=g - 2)
            return carry

        lax.fori_loop(0, (n_chunks - 4) // NBUF, group, 0)

        process(n_chunks - 2, (n_chunks - 2) % NBUF,
                prefetch_b=n_chunks % NBUF, sswait_g=n_chunks - 4)
        process(n_chunks - 1, (n_chunks - 1) % NBUF,
                prefetch_b=(n_chunks + 1) % NBUF, sswait_g=n_chunks - 3)
        s_copy(n_chunks - 2, (n_chunks - 2) % NBUF).wait()
        s_copy(n_chunks - 1, (n_chunks - 1) % NBUF).wait()

    return emb


def kernel(input, table):
    B0, B1 = input.shape
    B = B0 * B1
    idx = input.reshape(B).astype(jnp.int32)
    out = _make_kernel(B)(idx, table)
    return out.reshape(B0, B1, D_MODEL)
